# Initial kernel scaffold; baseline (speedup 1.0000x reference)
import jax
import jax.numpy as jnp
from jax.experimental import pallas as pl


def kernel(x, edge_index, W1, b1, W2, b2, W3, b3, Wg, ag_src, ag_dst, bg, f1w, f1b, f2w, f2b):
    raise NotImplementedError("write your pallas kernel here")


# speedup vs baseline: 21.9336x; 21.9336x over previous
import functools

import jax
import jax.numpy as jnp
from jax import lax
from jax.experimental import pallas as pl
from jax.experimental.pallas import tpu as pltpu
from jax.experimental.pallas import tpu_sc as plsc

f32 = jnp.float32
i32 = jnp.int32

N = 10000
NP = 10240
D = 128
E = 320000
ET = E + N
K = 128
NW = 32
CH = 81
EPT = K * CH
ET_PAD = EPT * NW
RPT = NP // 16
NZ = RPT // K
BN = 2560
GRID = NP // BN

MESH = plsc.VectorSubcoreMesh(
    core_axis_name="c", subcore_axis_name="s", num_cores=2, num_subcores=16
)

_SC_PARAMS = pltpu.CompilerParams(needs_layout_passes=False)


def _wid():
    cid = lax.axis_index("c")
    sid = lax.axis_index("s")
    return cid, sid, cid * 16 + sid


@functools.partial(
    pl.kernel,
    out_type=jax.ShapeDtypeStruct((2 * NP, D), f32),
    mesh=MESH,
    scratch_types=[
        pltpu.VMEM((K,), i32),
        pltpu.VMEM((K, D), f32),
        pltpu.VMEM_SHARED((NP, D), f32),
    ],
)
def _deg_k(dst_hbm, ones_hbm, z128_hbm, out_hbm, idx_v, buf_v, acc_sh):
    cid, sid, wid = _wid()
    pltpu.sync_copy(z128_hbm, buf_v)
    for z in range(NZ):
        pltpu.sync_copy(buf_v, acc_sh.at[pl.ds(sid * RPT + z * K, K)])
    pltpu.sync_copy(ones_hbm, buf_v)
    plsc.subcore_barrier()

    def body(ci, carry):
        base = wid * EPT + ci * K
        pltpu.sync_copy(dst_hbm.at[pl.ds(base, K)], idx_v)
        pltpu.sync_copy(buf_v, acc_sh.at[idx_v], add=True)
        return carry

    lax.fori_loop(0, CH, body, 0)
    plsc.subcore_barrier()
    for z in range(NZ):
        pltpu.sync_copy(acc_sh.at[pl.ds(sid * RPT + z * K, K)], buf_v)
        pltpu.sync_copy(
            buf_v, out_hbm.at[pl.ds(cid * NP + sid * RPT + z * K, K)])


@functools.partial(
    pl.kernel,
    out_type=jax.ShapeDtypeStruct((2 * NP, D), f32),
    mesh=MESH,
    scratch_types=[
        pltpu.VMEM((K,), i32),
        pltpu.VMEM((K,), i32),
        pltpu.VMEM((K, D), f32),
        pltpu.VMEM_SHARED((NP, D), f32),
        pltpu.SemaphoreType.DMA,
    ],
)
def _gcn_k(y_hbm, src_hbm, dst_hbm, z128_hbm, out_hbm, idxs_v, idxd_v, rows_v,
           acc_sh, sem):
    cid, sid, wid = _wid()
    pltpu.sync_copy(z128_hbm, rows_v)
    for z in range(NZ):
        pltpu.sync_copy(rows_v, acc_sh.at[pl.ds(sid * RPT + z * K, K)])
    plsc.subcore_barrier()

    def body(ci, carry):
        base = wid * EPT + ci * K
        pltpu.sync_copy(src_hbm.at[pl.ds(base, K)], idxs_v)
        pltpu.sync_copy(dst_hbm.at[pl.ds(base, K)], idxd_v)
        pltpu.async_copy(y_hbm.at[idxs_v], rows_v, sem).wait()
        pltpu.sync_copy(rows_v, acc_sh.at[idxd_v], add=True)
        return carry

    lax.fori_loop(0, CH, body, 0)
    plsc.subcore_barrier()
    for z in range(NZ):
        pltpu.sync_copy(acc_sh.at[pl.ds(sid * RPT + z * K, K)], rows_v)
        pltpu.sync_copy(
            rows_v, out_hbm.at[pl.ds(cid * NP + sid * RPT + z * K, K)])


@functools.partial(
    pl.kernel,
    out_type=[
        jax.ShapeDtypeStruct((ET_PAD,), f32),
        jax.ShapeDtypeStruct((ET_PAD,), f32),
        jax.ShapeDtypeStruct((NW * 16,), f32),
    ],
    mesh=MESH,
    scratch_types=[
        pltpu.VMEM((K,), i32),
        pltpu.VMEM((K,), i32),
        pltpu.VMEM((K,), f32),
        pltpu.VMEM((K,), f32),
        pltpu.VMEM((NP,), f32),
        pltpu.VMEM((NP,), f32),
        pltpu.VMEM((NP,), f32),
        pltpu.VMEM((NP,), f32),
        pltpu.VMEM((16,), f32),
    ],
    compiler_params=_SC_PARAMS,
)
def _gat1_k(als0_hbm, als1_hbm, ald0_hbm, ald1_hbm, src_hbm, dst_hbm,
            e0_hbm, e1_hbm, mx_hbm,
            idxs_v, idxd_v, e0_v, e1_v, als0_v, als1_v, ald0_v, ald1_v, mx_v):
    cid, sid, wid = _wid()
    pltpu.sync_copy(als0_hbm, als0_v)
    pltpu.sync_copy(als1_hbm, als1_v)
    pltpu.sync_copy(ald0_hbm, ald0_v)
    pltpu.sync_copy(ald1_hbm, ald1_v)

    def body(ci, m):
        base = wid * EPT + ci * K
        pltpu.sync_copy(src_hbm.at[pl.ds(base, K)], idxs_v)
        pltpu.sync_copy(dst_hbm.at[pl.ds(base, K)], idxd_v)
        for g in range(K // 16):
            sv = idxs_v[pl.ds(g * 16, 16)]
            dv = idxd_v[pl.ds(g * 16, 16)]
            z0 = plsc.load_gather(als0_v, [sv]) + plsc.load_gather(
                ald0_v, [dv])
            z1 = plsc.load_gather(als1_v, [sv]) + plsc.load_gather(
                ald1_v, [dv])
            e0 = jnp.where(z0 > 0, z0, 0.2 * z0)
            e1 = jnp.where(z1 > 0, z1, 0.2 * z1)
            e0_v[pl.ds(g * 16, 16)] = e0
            e1_v[pl.ds(g * 16, 16)] = e1
            m = jnp.maximum(m, jnp.maximum(e0, e1))
        pltpu.sync_copy(e0_v, e0_hbm.at[pl.ds(base, K)])
        pltpu.sync_copy(e1_v, e1_hbm.at[pl.ds(base, K)])
        return m

    m = lax.fori_loop(0, CH, body, jnp.full((16,), -1e30, f32))
    mx_v[...] = m
    pltpu.sync_copy(mx_v, mx_hbm.at[pl.ds(wid * 16, 16)])


def _lane_bcast(v, j):
    idx = jnp.full((16, 1), j, i32)
    return lax.gather(
        v, idx,
        lax.GatherDimensionNumbers(
            offset_dims=(), collapsed_slice_dims=(0,), start_index_map=(0,)),
        (1,), mode=lax.GatherScatterMode.PROMISE_IN_BOUNDS)


def _global_max(mxl_v):
    m = jnp.full((16,), -1e30, f32)
    for i in range(NW):
        m = jnp.maximum(m, mxl_v[pl.ds(i * 16, 16)])
    return jnp.max(m)


@functools.partial(
    pl.kernel,
    out_type=jax.ShapeDtypeStruct((2 * NP, D), f32),
    mesh=MESH,
    scratch_types=[
        pltpu.VMEM((K,), i32),
        pltpu.VMEM((K,), i32),
        pltpu.VMEM((K,), f32),
        pltpu.VMEM((K, D), f32),
        pltpu.VMEM((NW * 16,), f32),
        pltpu.VMEM_SHARED((NP, D), f32),
        pltpu.SemaphoreType.DMA,
    ],
    compiler_params=_SC_PARAMS,
)
def _gat2r_k(y_hbm, e_hbm, src_hbm, dst_hbm, mx_hbm, z128_hbm, out_hbm,
             idxs_v, idxd_v, e_v, rows_v, mxl_v, acc_sh, sem):
    cid, sid, wid = _wid()
    pltpu.sync_copy(z128_hbm, rows_v)
    for z in range(NZ):
        pltpu.sync_copy(rows_v, acc_sh.at[pl.ds(sid * RPT + z * K, K)])
    pltpu.sync_copy(mx_hbm, mxl_v)
    mglob = _global_max(mxl_v)
    plsc.subcore_barrier()

    def body(ci, carry):
        base = wid * EPT + ci * K
        pltpu.sync_copy(src_hbm.at[pl.ds(base, K)], idxs_v)
        pltpu.sync_copy(dst_hbm.at[pl.ds(base, K)], idxd_v)
        pltpu.sync_copy(e_hbm.at[pl.ds(base, K)], e_v)
        cp = pltpu.async_copy(y_hbm.at[idxs_v], rows_v, sem)
        for g in range(K // 16):
            wv = jnp.exp(e_v[pl.ds(g * 16, 16)] - mglob)
            e_v[pl.ds(g * 16, 16)] = wv
        cp.wait()
        for g in range(K // 16):
            wv = e_v[pl.ds(g * 16, 16)]
            for j in range(16):
                wb = _lane_bcast(wv, j)
                r = g * 16 + j
                for c in range(D // 16):
                    rows_v[r, pl.ds(c * 16, 16)] = (
                        rows_v[r, pl.ds(c * 16, 16)] * wb)
        pltpu.sync_copy(rows_v, acc_sh.at[idxd_v], add=True)
        return carry

    lax.fori_loop(0, CH, body, 0)
    plsc.subcore_barrier()
    for z in range(NZ):
        pltpu.sync_copy(acc_sh.at[pl.ds(sid * RPT + z * K, K)], rows_v)
        pltpu.sync_copy(
            rows_v, out_hbm.at[pl.ds(cid * NP + sid * RPT + z * K, K)])


@functools.partial(
    pl.kernel,
    out_type=jax.ShapeDtypeStruct((2 * NP, D), f32),
    mesh=MESH,
    scratch_types=[
        pltpu.VMEM((K,), i32),
        pltpu.VMEM((K,), f32),
        pltpu.VMEM((K, D), f32),
        pltpu.VMEM((NW * 16,), f32),
        pltpu.VMEM_SHARED((NP, D), f32),
    ],
    compiler_params=_SC_PARAMS,
)
def _gat2d_k(e_hbm, dst_hbm, mx_hbm, z128_hbm, den_hbm,
             idxd_v, e_v, wrow_v, mxl_v, den_sh):
    cid, sid, wid = _wid()
    pltpu.sync_copy(z128_hbm, wrow_v)
    for z in range(NZ):
        pltpu.sync_copy(wrow_v, den_sh.at[pl.ds(sid * RPT + z * K, K)])
    pltpu.sync_copy(mx_hbm, mxl_v)
    mglob = _global_max(mxl_v)
    plsc.subcore_barrier()

    def body(ci, carry):
        base = wid * EPT + ci * K
        pltpu.sync_copy(dst_hbm.at[pl.ds(base, K)], idxd_v)
        pltpu.sync_copy(e_hbm.at[pl.ds(base, K)], e_v)
        for g in range(K // 16):
            wv = jnp.exp(e_v[pl.ds(g * 16, 16)] - mglob)
            for j in range(16):
                wrow_v[g * 16 + j, pl.ds(0, 16)] = _lane_bcast(wv, j)
        pltpu.sync_copy(wrow_v, den_sh.at[idxd_v], add=True)
        return carry

    lax.fori_loop(0, CH, body, 0)
    plsc.subcore_barrier()
    for z in range(NZ):
        pltpu.sync_copy(den_sh.at[pl.ds(sid * RPT + z * K, K)], wrow_v)
        pltpu.sync_copy(
            wrow_v, den_hbm.at[pl.ds(cid * NP + sid * RPT + z * K, K)])


def _prep_body(x_ref, dp_ref, w_ref, y_ref, dinv_ref):
    deg = dp_ref[0, :, 0:1] + dp_ref[1, :, 0:1]
    dinv = jnp.where(deg > 0, 1.0 / jnp.sqrt(jnp.maximum(deg, 1e-12)), 0.0)
    dinvb = jnp.broadcast_to(dinv, (BN, D))
    xw = jnp.dot(x_ref[...], w_ref[...], preferred_element_type=f32)
    y_ref[...] = dinvb * xw
    dinv_ref[...] = dinvb


def _tc_prep(xp, dp, W1):
    return pl.pallas_call(
        _prep_body,
        grid=(GRID,),
        in_specs=[
            pl.BlockSpec((BN, D), lambda i: (i, 0)),
            pl.BlockSpec((2, BN, D), lambda i: (0, i, 0)),
            pl.BlockSpec((D, D), lambda i: (0, 0)),
        ],
        out_specs=[pl.BlockSpec((BN, D), lambda i: (i, 0))] * 2,
        out_shape=[jax.ShapeDtypeStruct((NP, D), f32)] * 2,
    )(xp, dp, W1)


def _mid_body(acc_ref, dinv_ref, b_ref, w_ref, y_ref):
    s = acc_ref[0] + acc_ref[1]
    h = jnp.maximum(dinv_ref[...] * s + b_ref[...], 0.0)
    y_ref[...] = dinv_ref[...] * jnp.dot(h, w_ref[...],
                                         preferred_element_type=f32)


def _tc_mid(acc, dinv, b, W):
    return pl.pallas_call(
        _mid_body,
        grid=(GRID,),
        in_specs=[
            pl.BlockSpec((2, BN, D), lambda i: (0, i, 0)),
            pl.BlockSpec((BN, D), lambda i: (i, 0)),
            pl.BlockSpec((1, D), lambda i: (0, 0)),
            pl.BlockSpec((D, D), lambda i: (0, 0)),
        ],
        out_specs=pl.BlockSpec((BN, D), lambda i: (i, 0)),
        out_shape=jax.ShapeDtypeStruct((NP, D), f32),
    )(acc, dinv, b, W)


def _gatprep_body(acc_ref, dinv_ref, b_ref, wg_ref, ams_ref, amd_ref,
                  xw_ref, als_ref, ald_ref):
    s = acc_ref[0] + acc_ref[1]
    h = jnp.maximum(dinv_ref[...] * s + b_ref[...], 0.0)
    xw = jnp.dot(h, wg_ref[...], preferred_element_type=f32)
    xw_ref[...] = xw
    als_ref[...] = jnp.dot(xw, ams_ref[...], preferred_element_type=f32)
    ald_ref[...] = jnp.dot(xw, amd_ref[...], preferred_element_type=f32)


def _tc_gatprep(acc, dinv, b, Wg, ams, amd):
    return pl.pallas_call(
        _gatprep_body,
        grid=(GRID,),
        in_specs=[
            pl.BlockSpec((2, BN, D), lambda i: (0, i, 0)),
            pl.BlockSpec((BN, D), lambda i: (i, 0)),
            pl.BlockSpec((1, D), lambda i: (0, 0)),
            pl.BlockSpec((D, 2 * D), lambda i: (0, 0)),
            pl.BlockSpec((2 * D, D), lambda i: (0, 0)),
            pl.BlockSpec((2 * D, D), lambda i: (0, 0)),
        ],
        out_specs=[
            pl.BlockSpec((BN, 2 * D), lambda i: (i, 0)),
            pl.BlockSpec((BN, D), lambda i: (i, 0)),
            pl.BlockSpec((BN, D), lambda i: (i, 0)),
        ],
        out_shape=[
            jax.ShapeDtypeStruct((NP, 2 * D), f32),
            jax.ShapeDtypeStruct((NP, D), f32),
            jax.ShapeDtypeStruct((NP, D), f32),
        ],
    )(acc, dinv, b, Wg, ams, amd)


def _final_body(go0_ref, gd0_ref, go1_ref, gd1_ref, bg_ref, f1w_ref, f1b_ref,
                f2w_ref, f2b_ref, out_ref, s_ref):
    i = pl.program_id(0)
    den0 = gd0_ref[0, :, 0:1] + gd0_ref[1, :, 0:1] + 1e-16
    den1 = gd1_ref[0, :, 0:1] + gd1_ref[1, :, 0:1] + 1e-16
    o0 = (go0_ref[0] + go0_ref[1]) / den0
    o1 = (go1_ref[0] + go1_ref[1]) / den1
    hg = jnp.maximum((o0 + o1) * 0.5 + bg_ref[...], 0.0)
    rows = lax.broadcasted_iota(i32, (BN, D), 0) + i * BN
    hg = jnp.where(rows < N, hg, 0.0)
    part = jnp.sum(hg, axis=0, keepdims=True)

    @pl.when(i == 0)
    def _():
        s_ref[...] = part

    @pl.when(i > 0)
    def _():
        s_ref[...] = s_ref[...] + part

    @pl.when(i == GRID - 1)
    def _():
        s = s_ref[...]
        g = s / float(N) + s
        t = jnp.maximum(
            jnp.dot(g, f1w_ref[...], preferred_element_type=f32)
            + f1b_ref[...], 0.0)
        o = jnp.dot(t, f2w_ref[...], preferred_element_type=f32) + f2b_ref[...]
        lanes = lax.broadcasted_iota(i32, (1, D), 1)
        om = jnp.where(lanes < 10, o, -jnp.inf)
        mx = jnp.max(om)
        lse = jnp.log(jnp.sum(jnp.exp(om - mx))) + mx
        out_ref[...] = om - lse


def _tc_final(go0, gd0, go1, gd1, bg, f1w, f1b, f2w_p, f2b_p):
    return pl.pallas_call(
        _final_body,
        grid=(GRID,),
        in_specs=[
            pl.BlockSpec((2, BN, D), lambda i: (0, i, 0)),
            pl.BlockSpec((2, BN, D), lambda i: (0, i, 0)),
            pl.BlockSpec((2, BN, D), lambda i: (0, i, 0)),
            pl.BlockSpec((2, BN, D), lambda i: (0, i, 0)),
            pl.BlockSpec((1, D), lambda i: (0, 0)),
            pl.BlockSpec((D, D), lambda i: (0, 0)),
            pl.BlockSpec((1, D), lambda i: (0, 0)),
            pl.BlockSpec((D, D), lambda i: (0, 0)),
            pl.BlockSpec((1, D), lambda i: (0, 0)),
        ],
        out_specs=pl.BlockSpec((1, D), lambda i: (0, 0)),
        out_shape=jax.ShapeDtypeStruct((1, D), f32),
        scratch_shapes=[pltpu.VMEM((1, D), f32)],
    )(go0, gd0, go1, gd1, bg, f1w, f1b, f2w_p, f2b_p)


def kernel(x, edge_index, W1, b1, W2, b2, W3, b3, Wg, ag_src, ag_dst, bg,
           f1w, f1b, f2w, f2b):
    loop = jnp.arange(N, dtype=i32)
    pad = jnp.full((ET_PAD - ET,), N, i32)
    src = jnp.concatenate([edge_index[0].astype(i32), loop, pad])
    dst = jnp.concatenate([edge_index[1].astype(i32), loop, pad])

    xp = jnp.zeros((NP, D), f32).at[:N].set(x)
    ones_kd = jnp.ones((K, D), f32)
    z128 = jnp.zeros((K, D), f32)

    dp = _deg_k(dst, ones_kd, z128).reshape(2, NP, D)
    y1, dinv = _tc_prep(xp, dp, W1)
    acc1 = _gcn_k(y1, src, dst, z128).reshape(2, NP, D)
    y2 = _tc_mid(acc1, dinv, b1.reshape(1, -1), W2)
    acc2 = _gcn_k(y2, src, dst, z128).reshape(2, NP, D)
    y3 = _tc_mid(acc2, dinv, b2.reshape(1, -1), W3)
    acc3 = _gcn_k(y3, src, dst, z128).reshape(2, NP, D)

    ams = jnp.zeros((2 * D, D), f32).at[:D, 0].set(ag_src[0]).at[D:, 1].set(
        ag_src[1])
    amd = jnp.zeros((2 * D, D), f32).at[:D, 0].set(ag_dst[0]).at[D:, 1].set(
        ag_dst[1])
    xw, alsf, aldf = _tc_gatprep(acc3, dinv, b3.reshape(1, -1), Wg, ams, amd)
    e0, e1, mx = _gat1_k(alsf[:, 0], alsf[:, 1], aldf[:, 0], aldf[:, 1],
                         src, dst)
    go0 = _gat2r_k(xw[:, :D], e0, src, dst, mx, z128).reshape(2, NP, D)
    go1 = _gat2r_k(xw[:, D:], e1, src, dst, mx, z128).reshape(2, NP, D)
    gd0 = _gat2d_k(e0, dst, mx, z128).reshape(2, NP, D)
    gd1 = _gat2d_k(e1, dst, mx, z128).reshape(2, NP, D)

    f2w_p = jnp.zeros((D, D), f32).at[:, :10].set(f2w)
    f2b_p = jnp.zeros((1, D), f32).at[0, :10].set(f2b)
    out = _tc_final(go0, gd0, go1, gd1, bg.reshape(1, -1), f1w,
                    f1b.reshape(1, -1), f2w_p, f2b_p)
    return out[:, :10]


# speedup vs baseline: 23.7829x; 1.0843x over previous
import functools

import jax
import jax.numpy as jnp
from jax import lax
from jax.experimental import pallas as pl
from jax.experimental.pallas import tpu as pltpu
from jax.experimental.pallas import tpu_sc as plsc

f32 = jnp.float32
i32 = jnp.int32

N = 10000
NP = 10240
D = 128
E = 320000
ET = E + N
K = 128
NW = 32
CH = 81
EPT = K * CH
ET_PAD = EPT * NW
RPT = NP // 16
NZ = RPT // K
K2 = 64
T2 = EPT // (2 * K2)
ET_EXT = ET_PAD + 2 * K2
BN = 2560
GRID = NP // BN

MESH = plsc.VectorSubcoreMesh(
    core_axis_name="c", subcore_axis_name="s", num_cores=2, num_subcores=16
)

_SC_PARAMS = pltpu.CompilerParams(needs_layout_passes=False)


def _wid():
    cid = lax.axis_index("c")
    sid = lax.axis_index("s")
    return cid, sid, cid * 16 + sid


@functools.partial(
    pl.kernel,
    out_type=jax.ShapeDtypeStruct((2 * NP, D), f32),
    mesh=MESH,
    scratch_types=[
        pltpu.VMEM((K,), i32),
        pltpu.VMEM((K, D), f32),
        pltpu.VMEM_SHARED((NP, D), f32),
    ],
)
def _deg_k(dst_hbm, ones_hbm, z128_hbm, out_hbm, idx_v, buf_v, acc_sh):
    cid, sid, wid = _wid()
    pltpu.sync_copy(z128_hbm, buf_v)
    for z in range(NZ):
        pltpu.sync_copy(buf_v, acc_sh.at[pl.ds(sid * RPT + z * K, K)])
    pltpu.sync_copy(ones_hbm, buf_v)
    plsc.subcore_barrier()

    def body(ci, carry):
        base = wid * EPT + ci * K
        pltpu.sync_copy(dst_hbm.at[pl.ds(base, K)], idx_v)
        pltpu.sync_copy(buf_v, acc_sh.at[idx_v], add=True)
        return carry

    lax.fori_loop(0, CH, body, 0)
    plsc.subcore_barrier()
    for z in range(NZ):
        pltpu.sync_copy(acc_sh.at[pl.ds(sid * RPT + z * K, K)], buf_v)
        pltpu.sync_copy(
            buf_v, out_hbm.at[pl.ds(cid * NP + sid * RPT + z * K, K)])


@functools.partial(
    pl.kernel,
    out_type=jax.ShapeDtypeStruct((2 * NP, D), f32),
    mesh=MESH,
    scratch_types=[
        pltpu.VMEM((K2,), i32),
        pltpu.VMEM((K2,), i32),
        pltpu.VMEM((K2,), i32),
        pltpu.VMEM((K2,), i32),
        pltpu.VMEM((K2, D), f32),
        pltpu.VMEM((K2, D), f32),
        pltpu.VMEM((K, D), f32),
        pltpu.VMEM_SHARED((NP, D), f32),
        pltpu.SemaphoreType.DMA,
        pltpu.SemaphoreType.DMA,
    ],
)
def _gcn_k(y_hbm, src_hbm, dst_hbm, z128_hbm, out_hbm,
           idxs0_v, idxd0_v, idxs1_v, idxd1_v, rows0_v, rows1_v, stg_v,
           acc_sh, sem0, sem1):
    cid, sid, wid = _wid()
    pltpu.sync_copy(z128_hbm, stg_v)
    for z in range(NZ):
        pltpu.sync_copy(stg_v, acc_sh.at[pl.ds(sid * RPT + z * K, K)])
    plsc.subcore_barrier()

    base0 = wid * EPT
    pltpu.sync_copy(src_hbm.at[pl.ds(base0, K2)], idxs0_v)
    pltpu.sync_copy(dst_hbm.at[pl.ds(base0, K2)], idxd0_v)
    pltpu.async_copy(y_hbm.at[idxs0_v], rows0_v, sem0)
    pltpu.sync_copy(src_hbm.at[pl.ds(base0 + K2, K2)], idxs1_v)
    pltpu.sync_copy(dst_hbm.at[pl.ds(base0 + K2, K2)], idxd1_v)
    pltpu.async_copy(y_hbm.at[idxs1_v], rows1_v, sem1)

    def body(t, carry):
        ba = wid * EPT + (2 * t + 2) * K2
        pltpu.make_async_copy(y_hbm.at[idxs0_v], rows0_v, sem0).wait()
        pltpu.sync_copy(rows0_v, acc_sh.at[idxd0_v], add=True)
        pltpu.sync_copy(src_hbm.at[pl.ds(ba, K2)], idxs0_v)
        pltpu.sync_copy(dst_hbm.at[pl.ds(ba, K2)], idxd0_v)
        pltpu.async_copy(y_hbm.at[idxs0_v], rows0_v, sem0)
        pltpu.make_async_copy(y_hbm.at[idxs1_v], rows1_v, sem1).wait()
        pltpu.sync_copy(rows1_v, acc_sh.at[idxd1_v], add=True)
        pltpu.sync_copy(src_hbm.at[pl.ds(ba + K2, K2)], idxs1_v)
        pltpu.sync_copy(dst_hbm.at[pl.ds(ba + K2, K2)], idxd1_v)
        pltpu.async_copy(y_hbm.at[idxs1_v], rows1_v, sem1)
        return carry

    lax.fori_loop(0, T2, body, 0)
    pltpu.make_async_copy(y_hbm.at[idxs0_v], rows0_v, sem0).wait()
    pltpu.make_async_copy(y_hbm.at[idxs1_v], rows1_v, sem1).wait()
    plsc.subcore_barrier()
    for z in range(NZ):
        pltpu.sync_copy(acc_sh.at[pl.ds(sid * RPT + z * K, K)], stg_v)
        pltpu.sync_copy(
            stg_v, out_hbm.at[pl.ds(cid * NP + sid * RPT + z * K, K)])


@functools.partial(
    pl.kernel,
    out_type=[
        jax.ShapeDtypeStruct((ET_EXT,), f32),
        jax.ShapeDtypeStruct((ET_EXT,), f32),
        jax.ShapeDtypeStruct((NW * 16,), f32),
    ],
    mesh=MESH,
    scratch_types=[
        pltpu.VMEM((K,), i32),
        pltpu.VMEM((K,), i32),
        pltpu.VMEM((K,), f32),
        pltpu.VMEM((K,), f32),
        pltpu.VMEM((NP,), f32),
        pltpu.VMEM((NP,), f32),
        pltpu.VMEM((NP,), f32),
        pltpu.VMEM((NP,), f32),
        pltpu.VMEM((16,), f32),
    ],
    compiler_params=_SC_PARAMS,
)
def _gat1_k(als0_hbm, als1_hbm, ald0_hbm, ald1_hbm, src_hbm, dst_hbm,
            e0_hbm, e1_hbm, mx_hbm,
            idxs_v, idxd_v, e0_v, e1_v, als0_v, als1_v, ald0_v, ald1_v, mx_v):
    cid, sid, wid = _wid()
    pltpu.sync_copy(als0_hbm, als0_v)
    pltpu.sync_copy(als1_hbm, als1_v)
    pltpu.sync_copy(ald0_hbm, ald0_v)
    pltpu.sync_copy(ald1_hbm, ald1_v)

    def body(ci, m):
        base = wid * EPT + ci * K
        pltpu.sync_copy(src_hbm.at[pl.ds(base, K)], idxs_v)
        pltpu.sync_copy(dst_hbm.at[pl.ds(base, K)], idxd_v)
        for g in range(K // 16):
            sv = idxs_v[pl.ds(g * 16, 16)]
            dv = idxd_v[pl.ds(g * 16, 16)]
            z0 = plsc.load_gather(als0_v, [sv]) + plsc.load_gather(
                ald0_v, [dv])
            z1 = plsc.load_gather(als1_v, [sv]) + plsc.load_gather(
                ald1_v, [dv])
            e0 = jnp.where(z0 > 0, z0, 0.2 * z0)
            e1 = jnp.where(z1 > 0, z1, 0.2 * z1)
            e0_v[pl.ds(g * 16, 16)] = e0
            e1_v[pl.ds(g * 16, 16)] = e1
            m = jnp.maximum(m, jnp.maximum(e0, e1))
        pltpu.sync_copy(e0_v, e0_hbm.at[pl.ds(base, K)])
        pltpu.sync_copy(e1_v, e1_hbm.at[pl.ds(base, K)])
        return m

    m = lax.fori_loop(0, CH, body, jnp.full((16,), -1e30, f32))
    mx_v[...] = m
    pltpu.sync_copy(mx_v, mx_hbm.at[pl.ds(wid * 16, 16)])


def _lane_bcast(v, j):
    idx = jnp.full((16, 1), j, i32)
    return lax.gather(
        v, idx,
        lax.GatherDimensionNumbers(
            offset_dims=(), collapsed_slice_dims=(0,), start_index_map=(0,)),
        (1,), mode=lax.GatherScatterMode.PROMISE_IN_BOUNDS)


def _global_max(mxl_v):
    m = jnp.full((16,), -1e30, f32)
    for i in range(NW):
        m = jnp.maximum(m, mxl_v[pl.ds(i * 16, 16)])
    return jnp.max(m)


def _wmul(rows_v, e_v, mglob):
    for g in range(K2 // 16):
        wv = jnp.exp(e_v[pl.ds(g * 16, 16)] - mglob)
        for j in range(16):
            wb = _lane_bcast(wv, j)
            r = g * 16 + j
            for c in range(D // 16):
                rows_v[r, pl.ds(c * 16, 16)] = (
                    rows_v[r, pl.ds(c * 16, 16)] * wb)


@functools.partial(
    pl.kernel,
    out_type=jax.ShapeDtypeStruct((2 * NP, D), f32),
    mesh=MESH,
    scratch_types=[
        pltpu.VMEM((K2,), i32),
        pltpu.VMEM((K2,), i32),
        pltpu.VMEM((K2,), i32),
        pltpu.VMEM((K2,), i32),
        pltpu.VMEM((K2,), f32),
        pltpu.VMEM((K2,), f32),
        pltpu.VMEM((K2, D), f32),
        pltpu.VMEM((K2, D), f32),
        pltpu.VMEM((K, D), f32),
        pltpu.VMEM((NW * 16,), f32),
        pltpu.VMEM_SHARED((NP, D), f32),
        pltpu.SemaphoreType.DMA,
        pltpu.SemaphoreType.DMA,
    ],
    compiler_params=_SC_PARAMS,
)
def _gat2r_k(y_hbm, e_hbm, src_hbm, dst_hbm, mx_hbm, z128_hbm, out_hbm,
             idxs0_v, idxd0_v, idxs1_v, idxd1_v, e0_v, e1_v, rows0_v, rows1_v,
             stg_v, mxl_v, acc_sh, sem0, sem1):
    cid, sid, wid = _wid()
    pltpu.sync_copy(z128_hbm, stg_v)
    for z in range(NZ):
        pltpu.sync_copy(stg_v, acc_sh.at[pl.ds(sid * RPT + z * K, K)])
    pltpu.sync_copy(mx_hbm, mxl_v)
    mglob = _global_max(mxl_v)
    plsc.subcore_barrier()

    base0 = wid * EPT
    pltpu.sync_copy(src_hbm.at[pl.ds(base0, K2)], idxs0_v)
    pltpu.sync_copy(dst_hbm.at[pl.ds(base0, K2)], idxd0_v)
    pltpu.sync_copy(e_hbm.at[pl.ds(base0, K2)], e0_v)
    pltpu.async_copy(y_hbm.at[idxs0_v], rows0_v, sem0)
    pltpu.sync_copy(src_hbm.at[pl.ds(base0 + K2, K2)], idxs1_v)
    pltpu.sync_copy(dst_hbm.at[pl.ds(base0 + K2, K2)], idxd1_v)
    pltpu.sync_copy(e_hbm.at[pl.ds(base0 + K2, K2)], e1_v)
    pltpu.async_copy(y_hbm.at[idxs1_v], rows1_v, sem1)

    def body(t, carry):
        ba = wid * EPT + (2 * t + 2) * K2
        pltpu.make_async_copy(y_hbm.at[idxs0_v], rows0_v, sem0).wait()
        _wmul(rows0_v, e0_v, mglob)
        pltpu.sync_copy(rows0_v, acc_sh.at[idxd0_v], add=True)
        pltpu.sync_copy(src_hbm.at[pl.ds(ba, K2)], idxs0_v)
        pltpu.sync_copy(dst_hbm.at[pl.ds(ba, K2)], idxd0_v)
        pltpu.sync_copy(e_hbm.at[pl.ds(ba, K2)], e0_v)
        pltpu.async_copy(y_hbm.at[idxs0_v], rows0_v, sem0)
        pltpu.make_async_copy(y_hbm.at[idxs1_v], rows1_v, sem1).wait()
        _wmul(rows1_v, e1_v, mglob)
        pltpu.sync_copy(rows1_v, acc_sh.at[idxd1_v], add=True)
        pltpu.sync_copy(src_hbm.at[pl.ds(ba + K2, K2)], idxs1_v)
        pltpu.sync_copy(dst_hbm.at[pl.ds(ba + K2, K2)], idxd1_v)
        pltpu.sync_copy(e_hbm.at[pl.ds(ba + K2, K2)], e1_v)
        pltpu.async_copy(y_hbm.at[idxs1_v], rows1_v, sem1)
        return carry

    lax.fori_loop(0, T2, body, 0)
    pltpu.make_async_copy(y_hbm.at[idxs0_v], rows0_v, sem0).wait()
    pltpu.make_async_copy(y_hbm.at[idxs1_v], rows1_v, sem1).wait()
    plsc.subcore_barrier()
    for z in range(NZ):
        pltpu.sync_copy(acc_sh.at[pl.ds(sid * RPT + z * K, K)], stg_v)
        pltpu.sync_copy(
            stg_v, out_hbm.at[pl.ds(cid * NP + sid * RPT + z * K, K)])


@functools.partial(
    pl.kernel,
    out_type=jax.ShapeDtypeStruct((2 * NP, D), f32),
    mesh=MESH,
    scratch_types=[
        pltpu.VMEM((K,), i32),
        pltpu.VMEM((K,), f32),
        pltpu.VMEM((K, D), f32),
        pltpu.VMEM((NW * 16,), f32),
        pltpu.VMEM_SHARED((NP, D), f32),
    ],
    compiler_params=_SC_PARAMS,
)
def _gat2d_k(e_hbm, dst_hbm, mx_hbm, z128_hbm, den_hbm,
             idxd_v, e_v, wrow_v, mxl_v, den_sh):
    cid, sid, wid = _wid()
    pltpu.sync_copy(z128_hbm, wrow_v)
    for z in range(NZ):
        pltpu.sync_copy(wrow_v, den_sh.at[pl.ds(sid * RPT + z * K, K)])
    pltpu.sync_copy(mx_hbm, mxl_v)
    mglob = _global_max(mxl_v)
    plsc.subcore_barrier()

    def body(ci, carry):
        base = wid * EPT + ci * K
        pltpu.sync_copy(dst_hbm.at[pl.ds(base, K)], idxd_v)
        pltpu.sync_copy(e_hbm.at[pl.ds(base, K)], e_v)
        for g in range(K // 16):
            wv = jnp.exp(e_v[pl.ds(g * 16, 16)] - mglob)
            for j in range(16):
                wrow_v[g * 16 + j, pl.ds(0, 16)] = _lane_bcast(wv, j)
        pltpu.sync_copy(wrow_v, den_sh.at[idxd_v], add=True)
        return carry

    lax.fori_loop(0, CH, body, 0)
    plsc.subcore_barrier()
    for z in range(NZ):
        pltpu.sync_copy(den_sh.at[pl.ds(sid * RPT + z * K, K)], wrow_v)
        pltpu.sync_copy(
            wrow_v, den_hbm.at[pl.ds(cid * NP + sid * RPT + z * K, K)])


def _prep_body(x_ref, dp_ref, w_ref, y_ref, dinv_ref):
    deg = dp_ref[0, :, 0:1] + dp_ref[1, :, 0:1]
    dinv = jnp.where(deg > 0, 1.0 / jnp.sqrt(jnp.maximum(deg, 1e-12)), 0.0)
    dinvb = jnp.broadcast_to(dinv, (BN, D))
    xw = jnp.dot(x_ref[...], w_ref[...], preferred_element_type=f32)
    y_ref[...] = dinvb * xw
    dinv_ref[...] = dinvb


def _tc_prep(xp, dp, W1):
    return pl.pallas_call(
        _prep_body,
        grid=(GRID,),
        in_specs=[
            pl.BlockSpec((BN, D), lambda i: (i, 0)),
            pl.BlockSpec((2, BN, D), lambda i: (0, i, 0)),
            pl.BlockSpec((D, D), lambda i: (0, 0)),
        ],
        out_specs=[pl.BlockSpec((BN, D), lambda i: (i, 0))] * 2,
        out_shape=[jax.ShapeDtypeStruct((NP, D), f32)] * 2,
    )(xp, dp, W1)


def _mid_body(acc_ref, dinv_ref, b_ref, w_ref, y_ref):
    s = acc_ref[0] + acc_ref[1]
    h = jnp.maximum(dinv_ref[...] * s + b_ref[...], 0.0)
    y_ref[...] = dinv_ref[...] * jnp.dot(h, w_ref[...],
                                         preferred_element_type=f32)


def _tc_mid(acc, dinv, b, W):
    return pl.pallas_call(
        _mid_body,
        grid=(GRID,),
        in_specs=[
            pl.BlockSpec((2, BN, D), lambda i: (0, i, 0)),
            pl.BlockSpec((BN, D), lambda i: (i, 0)),
            pl.BlockSpec((1, D), lambda i: (0, 0)),
            pl.BlockSpec((D, D), lambda i: (0, 0)),
        ],
        out_specs=pl.BlockSpec((BN, D), lambda i: (i, 0)),
        out_shape=jax.ShapeDtypeStruct((NP, D), f32),
    )(acc, dinv, b, W)


def _gatprep_body(acc_ref, dinv_ref, b_ref, wg_ref, ams_ref, amd_ref,
                  xw_ref, als_ref, ald_ref):
    s = acc_ref[0] + acc_ref[1]
    h = jnp.maximum(dinv_ref[...] * s + b_ref[...], 0.0)
    xw = jnp.dot(h, wg_ref[...], preferred_element_type=f32)
    xw_ref[...] = xw
    als_ref[...] = jnp.dot(xw, ams_ref[...], preferred_element_type=f32)
    ald_ref[...] = jnp.dot(xw, amd_ref[...], preferred_element_type=f32)


def _tc_gatprep(acc, dinv, b, Wg, ams, amd):
    return pl.pallas_call(
        _gatprep_body,
        grid=(GRID,),
        in_specs=[
            pl.BlockSpec((2, BN, D), lambda i: (0, i, 0)),
            pl.BlockSpec((BN, D), lambda i: (i, 0)),
            pl.BlockSpec((1, D), lambda i: (0, 0)),
            pl.BlockSpec((D, 2 * D), lambda i: (0, 0)),
            pl.BlockSpec((2 * D, D), lambda i: (0, 0)),
            pl.BlockSpec((2 * D, D), lambda i: (0, 0)),
        ],
        out_specs=[
            pl.BlockSpec((BN, 2 * D), lambda i: (i, 0)),
            pl.BlockSpec((BN, D), lambda i: (i, 0)),
            pl.BlockSpec((BN, D), lambda i: (i, 0)),
        ],
        out_shape=[
            jax.ShapeDtypeStruct((NP, 2 * D), f32),
            jax.ShapeDtypeStruct((NP, D), f32),
            jax.ShapeDtypeStruct((NP, D), f32),
        ],
    )(acc, dinv, b, Wg, ams, amd)


def _final_body(go0_ref, gd0_ref, go1_ref, gd1_ref, bg_ref, f1w_ref, f1b_ref,
                f2w_ref, f2b_ref, out_ref, s_ref):
    i = pl.program_id(0)
    den0 = gd0_ref[0, :, 0:1] + gd0_ref[1, :, 0:1] + 1e-16
    den1 = gd1_ref[0, :, 0:1] + gd1_ref[1, :, 0:1] + 1e-16
    o0 = (go0_ref[0] + go0_ref[1]) / den0
    o1 = (go1_ref[0] + go1_ref[1]) / den1
    hg = jnp.maximum((o0 + o1) * 0.5 + bg_ref[...], 0.0)
    rows = lax.broadcasted_iota(i32, (BN, D), 0) + i * BN
    hg = jnp.where(rows < N, hg, 0.0)
    part = jnp.sum(hg, axis=0, keepdims=True)

    @pl.when(i == 0)
    def _():
        s_ref[...] = part

    @pl.when(i > 0)
    def _():
        s_ref[...] = s_ref[...] + part

    @pl.when(i == GRID - 1)
    def _():
        s = s_ref[...]
        g = s / float(N) + s
        t = jnp.maximum(
            jnp.dot(g, f1w_ref[...], preferred_element_type=f32)
            + f1b_ref[...], 0.0)
        o = jnp.dot(t, f2w_ref[...], preferred_element_type=f32) + f2b_ref[...]
        lanes = lax.broadcasted_iota(i32, (1, D), 1)
        om = jnp.where(lanes < 10, o, -jnp.inf)
        mx = jnp.max(om)
        lse = jnp.log(jnp.sum(jnp.exp(om - mx))) + mx
        out_ref[...] = om - lse


def _tc_final(go0, gd0, go1, gd1, bg, f1w, f1b, f2w_p, f2b_p):
    return pl.pallas_call(
        _final_body,
        grid=(GRID,),
        in_specs=[
            pl.BlockSpec((2, BN, D), lambda i: (0, i, 0)),
            pl.BlockSpec((2, BN, D), lambda i: (0, i, 0)),
            pl.BlockSpec((2, BN, D), lambda i: (0, i, 0)),
            pl.BlockSpec((2, BN, D), lambda i: (0, i, 0)),
            pl.BlockSpec((1, D), lambda i: (0, 0)),
            pl.BlockSpec((D, D), lambda i: (0, 0)),
            pl.BlockSpec((1, D), lambda i: (0, 0)),
            pl.BlockSpec((D, D), lambda i: (0, 0)),
            pl.BlockSpec((1, D), lambda i: (0, 0)),
        ],
        out_specs=pl.BlockSpec((1, D), lambda i: (0, 0)),
        out_shape=jax.ShapeDtypeStruct((1, D), f32),
        scratch_shapes=[pltpu.VMEM((1, D), f32)],
    )(go0, gd0, go1, gd1, bg, f1w, f1b, f2w_p, f2b_p)


def kernel(x, edge_index, W1, b1, W2, b2, W3, b3, Wg, ag_src, ag_dst, bg,
           f1w, f1b, f2w, f2b):
    loop = jnp.arange(N, dtype=i32)
    pad = jnp.full((ET_EXT - ET,), N, i32)
    src = jnp.concatenate([edge_index[0].astype(i32), loop, pad])
    dst = jnp.concatenate([edge_index[1].astype(i32), loop, pad])

    xp = jnp.zeros((NP, D), f32).at[:N].set(x)
    ones_kd = jnp.ones((K, D), f32)
    z128 = jnp.zeros((K, D), f32)

    dp = _deg_k(dst, ones_kd, z128).reshape(2, NP, D)
    y1, dinv = _tc_prep(xp, dp, W1)
    acc1 = _gcn_k(y1, src, dst, z128).reshape(2, NP, D)
    y2 = _tc_mid(acc1, dinv, b1.reshape(1, -1), W2)
    acc2 = _gcn_k(y2, src, dst, z128).reshape(2, NP, D)
    y3 = _tc_mid(acc2, dinv, b2.reshape(1, -1), W3)
    acc3 = _gcn_k(y3, src, dst, z128).reshape(2, NP, D)

    ams = jnp.zeros((2 * D, D), f32).at[:D, 0].set(ag_src[0]).at[D:, 1].set(
        ag_src[1])
    amd = jnp.zeros((2 * D, D), f32).at[:D, 0].set(ag_dst[0]).at[D:, 1].set(
        ag_dst[1])
    xw, alsf, aldf = _tc_gatprep(acc3, dinv, b3.reshape(1, -1), Wg, ams, amd)
    e0, e1, mx = _gat1_k(alsf[:, 0], alsf[:, 1], aldf[:, 0], aldf[:, 1],
                         src, dst)
    go0 = _gat2r_k(xw[:, :D], e0, src, dst, mx, z128).reshape(2, NP, D)
    go1 = _gat2r_k(xw[:, D:], e1, src, dst, mx, z128).reshape(2, NP, D)
    gd0 = _gat2d_k(e0, dst, mx, z128).reshape(2, NP, D)
    gd1 = _gat2d_k(e1, dst, mx, z128).reshape(2, NP, D)

    f2w_p = jnp.zeros((D, D), f32).at[:, :10].set(f2w)
    f2b_p = jnp.zeros((1, D), f32).at[0, :10].set(f2b)
    out = _tc_final(go0, gd0, go1, gd1, bg.reshape(1, -1), f1w,
                    f1b.reshape(1, -1), f2w_p, f2b_p)
    return out[:, :10]
